# Initial kernel scaffold; baseline (speedup 1.0000x reference)
import functools

import jax
import jax.numpy as jnp
from jax import lax
from jax.experimental import pallas as pl
from jax.experimental.pallas import tpu as pltpu
from jax.experimental.pallas import tpu_sc as plsc

N_NODES = 10000
D = 128
N_EDGES = 320000
N_GRAPHS = 64

NC = 2
NS = 16
NW = NC * NS
E_PER_TILE = N_EDGES // NW
CHUNK = 125
NCHUNK = E_PER_TILE // CHUNK
ROWS_PER_TILE = N_NODES // NS
ZROWS = 125
HP = 10240
HSLC = HP // NS

_mesh = plsc.VectorSubcoreMesh(core_axis_name="c", subcore_axis_name="s")

BLK = 1000
NBLK = N_NODES // BLK


def _sc_degree(dst2d):
    @functools.partial(
        pl.kernel,
        out_type=jax.ShapeDtypeStruct((NC, HP), jnp.float32),
        mesh=_mesh,
        scratch_types=[
            pltpu.VMEM((NCHUNK, CHUNK), jnp.int32),
            pltpu.VMEM((128,), jnp.float32),
            pltpu.VMEM((HSLC,), jnp.float32),
            pltpu.VMEM_SHARED((HP,), jnp.float32),
        ],
    )
    def k(dst_hbm, out_hbm, didx, ones_v, zv, shist):
        cid = lax.axis_index("c")
        sid = lax.axis_index("s")
        wid = cid * NS + sid

        @pl.loop(0, 128, step=16)
        def _(i):
            ones_v[pl.ds(i, 16)] = jnp.full((16,), 1.0, jnp.float32)

        @pl.loop(0, HSLC, step=16)
        def _(i):
            zv[pl.ds(i, 16)] = jnp.zeros((16,), jnp.float32)

        pltpu.sync_copy(zv, shist.at[pl.ds(sid * HSLC, HSLC)])
        pltpu.sync_copy(dst_hbm.at[pl.ds(wid * NCHUNK, NCHUNK)], didx)
        plsc.subcore_barrier()

        @pl.loop(0, NCHUNK)
        def _(j):
            pltpu.sync_copy(ones_v.at[pl.ds(0, CHUNK)],
                            shist.at[didx.at[j]], add=True)

        plsc.subcore_barrier()
        pltpu.sync_copy(shist.at[pl.ds(sid * HSLC, HSLC)],
                        out_hbm.at[cid, pl.ds(sid * HSLC, HSLC)])

    return k(dst2d)


def _sc_aggregate(y, src2d, dst2d):
    @functools.partial(
        pl.kernel,
        out_type=jax.ShapeDtypeStruct((NC, N_NODES, D), jnp.float32),
        mesh=_mesh,
        scratch_types=[
            pltpu.VMEM((NCHUNK, CHUNK), jnp.int32),
            pltpu.VMEM((NCHUNK, CHUNK), jnp.int32),
            pltpu.VMEM((CHUNK, D), jnp.float32),
            pltpu.VMEM((CHUNK, D), jnp.float32),
            pltpu.VMEM((ZROWS, D), jnp.float32),
            pltpu.VMEM_SHARED((N_NODES, D), jnp.float32),
            pltpu.SemaphoreType.DMA,
            pltpu.SemaphoreType.DMA,
        ],
    )
    def k(y_hbm, src_hbm, dst_hbm, out_hbm,
          sidx, didx, rows0, rows1, zbuf, acc, g0, g1):
        cid = lax.axis_index("c")
        sid = lax.axis_index("s")
        wid = cid * NS + sid

        @pl.loop(0, ZROWS)
        def _(r):
            @pl.loop(0, D, step=16)
            def _(c):
                zbuf[r, pl.ds(c, 16)] = jnp.zeros((16,), jnp.float32)

        base = sid * ROWS_PER_TILE

        @pl.loop(0, ROWS_PER_TILE, step=ZROWS)
        def _(o):
            pltpu.sync_copy(zbuf, acc.at[pl.ds(base + o, ZROWS)])

        pltpu.sync_copy(src_hbm.at[pl.ds(wid * NCHUNK, NCHUNK)], sidx)
        pltpu.sync_copy(dst_hbm.at[pl.ds(wid * NCHUNK, NCHUNK)], didx)
        plsc.subcore_barrier()

        pltpu.async_copy(y_hbm.at[sidx.at[0]], rows0, g0)

        @pl.loop(0, NCHUNK, step=2)
        def _(j):
            pltpu.make_async_copy(y_hbm.at[sidx.at[j]], rows0, g0).wait()
            pltpu.async_copy(y_hbm.at[sidx.at[j + 1]], rows1, g1)
            pltpu.sync_copy(rows0, acc.at[didx.at[j]], add=True)
            pltpu.make_async_copy(y_hbm.at[sidx.at[j + 1]], rows1, g1).wait()

            @pl.when(j + 2 < NCHUNK)
            def _():
                pltpu.async_copy(y_hbm.at[sidx.at[j + 2]], rows0, g0)

            pltpu.sync_copy(rows1, acc.at[didx.at[j + 1]], add=True)

        plsc.subcore_barrier()

        @pl.loop(0, ROWS_PER_TILE, step=ZROWS)
        def _(o):
            pltpu.sync_copy(acc.at[pl.ds(base + o, ZROWS)],
                            out_hbm.at[cid, pl.ds(base + o, ZROWS)])

    return k(y, src2d, dst2d)


def _tc_matmul(x, W):
    def body(x_ref, w_ref, o_ref):
        o_ref[...] = jnp.dot(x_ref[...], w_ref[...],
                             preferred_element_type=jnp.float32)

    return pl.pallas_call(
        body,
        grid=(NBLK,),
        in_specs=[pl.BlockSpec((BLK, D), lambda i: (i, 0)),
                  pl.BlockSpec((D, D), lambda i: (0, 0))],
        out_specs=pl.BlockSpec((BLK, D), lambda i: (i, 0)),
        out_shape=jax.ShapeDtypeStruct((N_NODES, D), jnp.float32),
    )(x, W)


def _tc_scale(xw, dpt):
    def body(xw_ref, dp_ref, y_ref, dinv_ref):
        deg = 1.0 + dp_ref[:, 0:1] + dp_ref[:, 1:2]
        dinv = lax.rsqrt(deg)
        dinv_ref[...] = dinv
        y_ref[...] = xw_ref[...] * dinv

    return pl.pallas_call(
        body,
        grid=(NBLK,),
        in_specs=[pl.BlockSpec((BLK, D), lambda i: (i, 0)),
                  pl.BlockSpec((BLK, 2), lambda i: (i, 0))],
        out_specs=[pl.BlockSpec((BLK, D), lambda i: (i, 0)),
                   pl.BlockSpec((BLK, 1), lambda i: (i, 0))],
        out_shape=[jax.ShapeDtypeStruct((N_NODES, D), jnp.float32),
                   jax.ShapeDtypeStruct((N_NODES, 1), jnp.float32)],
    )(xw, dpt)


def _tc_mid(a0, a1, y1, dinv, b1, W2):
    def body(a0_ref, a1_ref, y1_ref, dinv_ref, b1_ref, w2_ref, y2_ref):
        dinv = dinv_ref[...]
        h = (a0_ref[...] + a1_ref[...] + y1_ref[...]) * dinv + b1_ref[...]
        h = jnp.maximum(h, 0.0)
        y2_ref[...] = jnp.dot(h, w2_ref[...],
                              preferred_element_type=jnp.float32) * dinv

    return pl.pallas_call(
        body,
        grid=(NBLK,),
        in_specs=[pl.BlockSpec((BLK, D), lambda i: (i, 0)),
                  pl.BlockSpec((BLK, D), lambda i: (i, 0)),
                  pl.BlockSpec((BLK, D), lambda i: (i, 0)),
                  pl.BlockSpec((BLK, 1), lambda i: (i, 0)),
                  pl.BlockSpec((1, D), lambda i: (0, 0)),
                  pl.BlockSpec((D, D), lambda i: (0, 0))],
        out_specs=pl.BlockSpec((BLK, D), lambda i: (i, 0)),
        out_shape=jax.ShapeDtypeStruct((N_NODES, D), jnp.float32),
    )(a0, a1, y1, dinv, b1, W2)


def _tc_pool(a0, a1, y2, dinv, b2, bcol):
    def body(a0_ref, a1_ref, y2_ref, dinv_ref, b2_ref, b_ref, p_ref):
        i = pl.program_id(0)

        @pl.when(i == 0)
        def _():
            p_ref[...] = jnp.full((N_GRAPHS, D), -jnp.inf, jnp.float32)

        h = ((a0_ref[...] + a1_ref[...] + y2_ref[...]) * dinv_ref[...]
             + b2_ref[...])
        b = b_ref[...]

        def upd(g, carry):
            m = jnp.max(jnp.where(b == g, h, -jnp.inf), axis=0, keepdims=True)
            p_ref[pl.ds(g, 1), :] = jnp.maximum(p_ref[pl.ds(g, 1), :], m)
            return carry

        lax.fori_loop(0, N_GRAPHS, upd, 0)

    return pl.pallas_call(
        body,
        grid=(NBLK,),
        in_specs=[pl.BlockSpec((BLK, D), lambda i: (i, 0)),
                  pl.BlockSpec((BLK, D), lambda i: (i, 0)),
                  pl.BlockSpec((BLK, D), lambda i: (i, 0)),
                  pl.BlockSpec((BLK, 1), lambda i: (i, 0)),
                  pl.BlockSpec((1, D), lambda i: (0, 0)),
                  pl.BlockSpec((BLK, 1), lambda i: (i, 0))],
        out_specs=pl.BlockSpec((N_GRAPHS, D), lambda i: (0, 0)),
        out_shape=jax.ShapeDtypeStruct((N_GRAPHS, D), jnp.float32),
    )(a0, a1, y2, dinv, b2, bcol)


def _tc_dec(pooled, Wd, bd):
    CBLK = 1000

    def body(p_ref, wd_ref, bd_ref, o_ref):
        o_ref[...] = jnp.dot(p_ref[...], wd_ref[...],
                             preferred_element_type=jnp.float32) + bd_ref[...]

    return pl.pallas_call(
        body,
        grid=(N_NODES // CBLK,),
        in_specs=[pl.BlockSpec((N_GRAPHS, D), lambda i: (0, 0)),
                  pl.BlockSpec((D, CBLK), lambda i: (0, i)),
                  pl.BlockSpec((1, CBLK), lambda i: (0, i))],
        out_specs=pl.BlockSpec((N_GRAPHS, CBLK), lambda i: (0, i)),
        out_shape=jax.ShapeDtypeStruct((N_GRAPHS, N_NODES), jnp.float32),
    )(pooled, Wd, bd)


def kernel(x, edge_index, batch, W1, b1, W2, b2, Wd, bd):
    src2d = edge_index[0].reshape(N_EDGES // CHUNK, CHUNK)
    dst2d = edge_index[1].reshape(N_EDGES // CHUNK, CHUNK)

    degp = _sc_degree(dst2d)
    xw1 = _tc_matmul(x, W1)
    dpt = jnp.transpose(degp[:, :N_NODES])
    y1, dinv = _tc_scale(xw1, dpt)

    agg1 = _sc_aggregate(y1, src2d, dst2d)
    y2 = _tc_mid(agg1[0], agg1[1], y1, dinv, b1.reshape(1, D), W2)

    agg2 = _sc_aggregate(y2, src2d, dst2d)
    pooled = _tc_pool(agg2[0], agg2[1], y2, dinv, b2.reshape(1, D),
                      batch.reshape(N_NODES, 1))
    return _tc_dec(pooled, Wd, bd.reshape(1, N_NODES))


# speedup vs baseline: 20.5432x; 20.5432x over previous
import functools

import jax
import jax.numpy as jnp
from jax import lax
from jax.experimental import pallas as pl
from jax.experimental.pallas import tpu as pltpu
from jax.experimental.pallas import tpu_sc as plsc

N_NODES = 10000
D = 128
N_EDGES = 320000
N_GRAPHS = 64

NC = 2
NS = 16
NW = NC * NS
E_PER_TILE = N_EDGES // NW
CHUNK = 125
NCHUNK = E_PER_TILE // CHUNK
NPAD = 10240
ROWS_PER_TILE = NPAD // NS
ZROWS = 128
HP = 10240
HSLC = HP // NS

_mesh = plsc.VectorSubcoreMesh(core_axis_name="c", subcore_axis_name="s")

BLK = 1000
NBLK = N_NODES // BLK


def _sc_degree(dst2d):
    @functools.partial(
        pl.kernel,
        out_type=jax.ShapeDtypeStruct((NC * HP,), jnp.float32),
        mesh=_mesh,
        scratch_types=[
            pltpu.VMEM((NCHUNK, CHUNK), jnp.int32),
            pltpu.VMEM((128,), jnp.float32),
            pltpu.VMEM((HSLC,), jnp.float32),
            pltpu.VMEM_SHARED((HP,), jnp.float32),
        ],
    )
    def k(dst_hbm, out_hbm, didx, ones_v, zv, shist):
        cid = lax.axis_index("c")
        sid = lax.axis_index("s")
        wid = cid * NS + sid

        @pl.loop(0, 128, step=16)
        def _(i):
            ones_v[pl.ds(i, 16)] = jnp.full((16,), 1.0, jnp.float32)

        @pl.loop(0, HSLC, step=16)
        def _(i):
            zv[pl.ds(i, 16)] = jnp.zeros((16,), jnp.float32)

        pltpu.sync_copy(zv, shist.at[pl.ds(sid * HSLC, HSLC)])
        pltpu.sync_copy(dst_hbm.at[pl.ds(wid * NCHUNK, NCHUNK)], didx)
        plsc.subcore_barrier()

        @pl.loop(0, NCHUNK)
        def _(j):
            pltpu.sync_copy(ones_v.at[pl.ds(0, CHUNK)],
                            shist.at[didx.at[j]], add=True)

        plsc.subcore_barrier()
        pltpu.sync_copy(shist.at[pl.ds(sid * HSLC, HSLC)],
                        out_hbm.at[pl.ds(cid * HP + sid * HSLC, HSLC)])

    return k(dst2d)


GRP = 8
NGRP = NCHUNK // GRP


def _sc_aggregate(y, src2d, dst2d):
    @functools.partial(
        pl.kernel,
        out_type=jax.ShapeDtypeStruct((NC * NPAD, D), jnp.float32),
        mesh=_mesh,
        scratch_types=[
            pltpu.VMEM((GRP, CHUNK), jnp.int32),
            pltpu.VMEM((GRP, CHUNK), jnp.int32),
            pltpu.VMEM((GRP, CHUNK), jnp.int32),
            pltpu.VMEM((GRP, CHUNK), jnp.int32),
            pltpu.VMEM((CHUNK, D), jnp.float32),
            pltpu.VMEM((CHUNK, D), jnp.float32),
            pltpu.VMEM_SHARED((NPAD, D), jnp.float32),
            pltpu.SemaphoreType.DMA,
            pltpu.SemaphoreType.DMA,
            pltpu.SemaphoreType.DMA,
            pltpu.SemaphoreType.DMA,
        ],
    )
    def k(y_hbm, src_hbm, dst_hbm, out_hbm,
          sr0, sr1, dr0, dr1, rows0, rows1, acc, is0, is1, gs0, gs1):
        cid = lax.axis_index("c")
        sid = lax.axis_index("s")
        wid = cid * NS + sid
        srings, drings = (sr0, sr1), (dr0, dr1)
        rows, gsems, isems = (rows0, rows1), (gs0, gs1), (is0, is1)
        brow = wid * NCHUNK

        def idx_start(grp, s):
            pltpu.async_copy(src_hbm.at[pl.ds(brow + grp * GRP, GRP)],
                             srings[s], isems[s])
            pltpu.async_copy(dst_hbm.at[pl.ds(brow + grp * GRP, GRP)],
                             drings[s], isems[s])

        def idx_wait(grp, s):
            pltpu.make_async_copy(src_hbm.at[pl.ds(brow + grp * GRP, GRP)],
                                  srings[s], isems[s]).wait()
            pltpu.make_async_copy(dst_hbm.at[pl.ds(brow + grp * GRP, GRP)],
                                  drings[s], isems[s]).wait()

        def gather_start(s, m, rb):
            pltpu.async_copy(y_hbm.at[srings[s].at[m]], rows[rb], gsems[rb])

        def gather_wait(s, m, rb):
            pltpu.make_async_copy(y_hbm.at[srings[s].at[m]], rows[rb],
                                  gsems[rb]).wait()

        @pl.loop(0, CHUNK)
        def _(r):
            @pl.loop(0, D, step=16)
            def _(c):
                rows0[r, pl.ds(c, 16)] = jnp.zeros((16,), jnp.float32)

        zbase = sid * ROWS_PER_TILE

        @pl.loop(0, 5 * CHUNK, step=CHUNK)
        def _(o):
            pltpu.sync_copy(rows0, acc.at[pl.ds(zbase + o, CHUNK)])

        pltpu.sync_copy(rows0.at[pl.ds(0, ROWS_PER_TILE - 5 * CHUNK)],
                        acc.at[pl.ds(zbase + 5 * CHUNK,
                                     ROWS_PER_TILE - 5 * CHUNK)])
        plsc.subcore_barrier()

        idx_start(0, 0)
        idx_wait(0, 0)
        gather_start(0, 0, 0)
        idx_start(1, 1)

        def group_body(g, s):
            for m in range(GRP):
                rb = m % 2
                gather_wait(s, m, rb)
                if m < GRP - 1:
                    gather_start(s, m + 1, 1 - rb)
                    pltpu.sync_copy(rows[rb], acc.at[drings[s].at[m]],
                                    add=True)
                else:
                    @pl.when(g + 1 < NGRP)
                    def _():
                        idx_wait(g + 1, 1 - s)
                        gather_start(1 - s, 0, 1 - rb)

                    pltpu.sync_copy(rows[rb], acc.at[drings[s].at[m]],
                                    add=True)

                    @pl.when(g + 2 < NGRP)
                    def _():
                        idx_start(g + 2, s)

        @pl.loop(0, NGRP, step=2)
        def _(g):
            group_body(g, 0)
            group_body(g + 1, 1)

        plsc.subcore_barrier()

        @pl.loop(0, ROWS_PER_TILE, step=ZROWS)
        def _(o):
            pltpu.sync_copy(acc.at[pl.ds(zbase + o, ZROWS)],
                            out_hbm.at[pl.ds(cid * NPAD + zbase + o, ZROWS)])

    return k(y, src2d, dst2d)


def _tc_matmul(x, W):
    def body(x_ref, w_ref, o_ref):
        o_ref[...] = jnp.dot(x_ref[...], w_ref[...],
                             preferred_element_type=jnp.float32)

    return pl.pallas_call(
        body,
        grid=(NBLK,),
        in_specs=[pl.BlockSpec((BLK, D), lambda i: (i, 0)),
                  pl.BlockSpec((D, D), lambda i: (0, 0))],
        out_specs=pl.BlockSpec((BLK, D), lambda i: (i, 0)),
        out_shape=jax.ShapeDtypeStruct((N_NODES, D), jnp.float32),
    )(x, W)


def _tc_scale(xw, dpt):
    def body(xw_ref, dp_ref, y_ref, dinv_ref):
        deg = 1.0 + dp_ref[:, 0:1] + dp_ref[:, 1:2]
        dinv = lax.rsqrt(deg)
        dinv_ref[...] = dinv
        y_ref[...] = xw_ref[...] * dinv

    return pl.pallas_call(
        body,
        grid=(NBLK,),
        in_specs=[pl.BlockSpec((BLK, D), lambda i: (i, 0)),
                  pl.BlockSpec((BLK, 2), lambda i: (i, 0))],
        out_specs=[pl.BlockSpec((BLK, D), lambda i: (i, 0)),
                   pl.BlockSpec((BLK, 1), lambda i: (i, 0))],
        out_shape=[jax.ShapeDtypeStruct((N_NODES, D), jnp.float32),
                   jax.ShapeDtypeStruct((N_NODES, 1), jnp.float32)],
    )(xw, dpt)


def _tc_mid(a0, a1, y1, dinv, b1, W2):
    def body(a0_ref, a1_ref, y1_ref, dinv_ref, b1_ref, w2_ref, y2_ref):
        dinv = dinv_ref[...]
        h = (a0_ref[...] + a1_ref[...] + y1_ref[...]) * dinv + b1_ref[...]
        h = jnp.maximum(h, 0.0)
        y2_ref[...] = jnp.dot(h, w2_ref[...],
                              preferred_element_type=jnp.float32) * dinv

    return pl.pallas_call(
        body,
        grid=(NBLK,),
        in_specs=[pl.BlockSpec((BLK, D), lambda i: (i, 0)),
                  pl.BlockSpec((BLK, D), lambda i: (i, 0)),
                  pl.BlockSpec((BLK, D), lambda i: (i, 0)),
                  pl.BlockSpec((BLK, 1), lambda i: (i, 0)),
                  pl.BlockSpec((1, D), lambda i: (0, 0)),
                  pl.BlockSpec((D, D), lambda i: (0, 0))],
        out_specs=pl.BlockSpec((BLK, D), lambda i: (i, 0)),
        out_shape=jax.ShapeDtypeStruct((N_NODES, D), jnp.float32),
    )(a0, a1, y1, dinv, b1, W2)


def _tc_pool(a0, a1, y2, dinv, b2, bcol):
    def body(a0_ref, a1_ref, y2_ref, dinv_ref, b2_ref, b_ref, p_ref):
        i = pl.program_id(0)

        @pl.when(i == 0)
        def _():
            p_ref[...] = jnp.full((N_GRAPHS, D), -jnp.inf, jnp.float32)

        h = ((a0_ref[...] + a1_ref[...] + y2_ref[...]) * dinv_ref[...]
             + b2_ref[...])
        b = b_ref[...]

        def upd(g, carry):
            m = jnp.max(jnp.where(b == g, h, -jnp.inf), axis=0, keepdims=True)
            p_ref[pl.ds(g, 1), :] = jnp.maximum(p_ref[pl.ds(g, 1), :], m)
            return carry

        lax.fori_loop(0, N_GRAPHS, upd, 0)

    return pl.pallas_call(
        body,
        grid=(NBLK,),
        in_specs=[pl.BlockSpec((BLK, D), lambda i: (i, 0)),
                  pl.BlockSpec((BLK, D), lambda i: (i, 0)),
                  pl.BlockSpec((BLK, D), lambda i: (i, 0)),
                  pl.BlockSpec((BLK, 1), lambda i: (i, 0)),
                  pl.BlockSpec((1, D), lambda i: (0, 0)),
                  pl.BlockSpec((BLK, 1), lambda i: (i, 0))],
        out_specs=pl.BlockSpec((N_GRAPHS, D), lambda i: (0, 0)),
        out_shape=jax.ShapeDtypeStruct((N_GRAPHS, D), jnp.float32),
    )(a0, a1, y2, dinv, b2, bcol)


def _tc_dec(pooled, Wd, bd):
    NP = Wd.shape[1]
    CBLK = 1024

    def body(p_ref, wd_ref, bd_ref, o_ref):
        o_ref[...] = jnp.dot(p_ref[...], wd_ref[...],
                             preferred_element_type=jnp.float32) + bd_ref[...]

    return pl.pallas_call(
        body,
        grid=(NP // CBLK,),
        in_specs=[pl.BlockSpec((N_GRAPHS, D), lambda i: (0, 0)),
                  pl.BlockSpec((D, CBLK), lambda i: (0, i)),
                  pl.BlockSpec((1, CBLK), lambda i: (0, i))],
        out_specs=pl.BlockSpec((N_GRAPHS, CBLK), lambda i: (0, i)),
        out_shape=jax.ShapeDtypeStruct((N_GRAPHS, NP), jnp.float32),
    )(pooled, Wd, bd)


def kernel(x, edge_index, batch, W1, b1, W2, b2, Wd, bd):
    src2d = edge_index[0].reshape(N_EDGES // CHUNK, CHUNK)
    dst2d = edge_index[1].reshape(N_EDGES // CHUNK, CHUNK)

    degp = _sc_degree(dst2d).reshape(NC, HP)
    xw1 = _tc_matmul(x, W1)
    dpt = jnp.transpose(degp[:, :N_NODES])
    y1, dinv = _tc_scale(xw1, dpt)

    agg1 = _sc_aggregate(y1, src2d, dst2d).reshape(NC, NPAD, D)
    y2 = _tc_mid(agg1[0, :N_NODES], agg1[1, :N_NODES], y1, dinv,
                 b1.reshape(1, D), W2)

    agg2 = _sc_aggregate(y2, src2d, dst2d).reshape(NC, NPAD, D)
    pooled = _tc_pool(agg2[0, :N_NODES], agg2[1, :N_NODES], y2, dinv,
                      b2.reshape(1, D), batch.reshape(N_NODES, 1))

    NP = 10240
    Wd_p = jnp.pad(Wd, ((0, 0), (0, NP - N_NODES)))
    bd_p = jnp.pad(bd.reshape(1, N_NODES), ((0, 0), (0, NP - N_NODES)))
    return _tc_dec(pooled, Wd_p, bd_p)[:, :N_NODES]


# speedup vs baseline: 26.5304x; 1.2914x over previous
import functools

import jax
import jax.numpy as jnp
from jax import lax
from jax.experimental import pallas as pl
from jax.experimental.pallas import tpu as pltpu
from jax.experimental.pallas import tpu_sc as plsc

N_NODES = 10000
D = 128
N_EDGES = 320000
N_GRAPHS = 64

NC = 2
NS = 16
NW = NC * NS
E_PER_TILE = N_EDGES // NW
CHUNK = 125
NCHUNK = E_PER_TILE // CHUNK
NPAD = 10240
ROWS_PER_TILE = NPAD // NS
ZROWS = 128
HP = 10240
HSLC = HP // NS

_mesh = plsc.VectorSubcoreMesh(core_axis_name="c", subcore_axis_name="s")

BLK = 1024
NBLK = NPAD // BLK


def _sc_degree(dst2d):
    @functools.partial(
        pl.kernel,
        out_type=jax.ShapeDtypeStruct((NC * HP,), jnp.float32),
        mesh=_mesh,
        scratch_types=[
            pltpu.VMEM((NCHUNK, CHUNK), jnp.int32),
            pltpu.VMEM((128,), jnp.float32),
            pltpu.VMEM((HSLC,), jnp.float32),
            pltpu.VMEM_SHARED((HP,), jnp.float32),
        ],
    )
    def k(dst_hbm, out_hbm, didx, ones_v, zv, shist):
        cid = lax.axis_index("c")
        sid = lax.axis_index("s")
        wid = cid * NS + sid

        @pl.loop(0, 128, step=16)
        def _(i):
            ones_v[pl.ds(i, 16)] = jnp.full((16,), 1.0, jnp.float32)

        @pl.loop(0, HSLC, step=16)
        def _(i):
            zv[pl.ds(i, 16)] = jnp.zeros((16,), jnp.float32)

        pltpu.sync_copy(zv, shist.at[pl.ds(sid * HSLC, HSLC)])
        pltpu.sync_copy(dst_hbm.at[pl.ds(wid * NCHUNK, NCHUNK)], didx)
        plsc.subcore_barrier()

        @pl.loop(0, NCHUNK)
        def _(j):
            pltpu.sync_copy(ones_v.at[pl.ds(0, CHUNK)],
                            shist.at[didx.at[j]], add=True)

        plsc.subcore_barrier()
        pltpu.sync_copy(shist.at[pl.ds(sid * HSLC, HSLC)],
                        out_hbm.at[pl.ds(cid * HP + sid * HSLC, HSLC)])

    return k(dst2d)


GRP = 8
NGRP = NCHUNK // GRP


def _sc_aggregate(y, src2d, dst2d):
    @functools.partial(
        pl.kernel,
        out_type=jax.ShapeDtypeStruct((NC * NPAD, D), jnp.float32),
        mesh=_mesh,
        scratch_types=[
            pltpu.VMEM((GRP, CHUNK), jnp.int32),
            pltpu.VMEM((GRP, CHUNK), jnp.int32),
            pltpu.VMEM((GRP, CHUNK), jnp.int32),
            pltpu.VMEM((GRP, CHUNK), jnp.int32),
            pltpu.VMEM((CHUNK, D), jnp.float32),
            pltpu.VMEM((CHUNK, D), jnp.float32),
            pltpu.VMEM_SHARED((NPAD, D), jnp.float32),
            pltpu.SemaphoreType.DMA,
            pltpu.SemaphoreType.DMA,
            pltpu.SemaphoreType.DMA,
            pltpu.SemaphoreType.DMA,
        ],
    )
    def k(y_hbm, src_hbm, dst_hbm, out_hbm,
          sr0, sr1, dr0, dr1, rows0, rows1, acc, is0, is1, gs0, gs1):
        cid = lax.axis_index("c")
        sid = lax.axis_index("s")
        wid = cid * NS + sid
        srings, drings = (sr0, sr1), (dr0, dr1)
        rows, gsems, isems = (rows0, rows1), (gs0, gs1), (is0, is1)
        brow = wid * NCHUNK

        def idx_start(grp, s):
            pltpu.async_copy(src_hbm.at[pl.ds(brow + grp * GRP, GRP)],
                             srings[s], isems[s])
            pltpu.async_copy(dst_hbm.at[pl.ds(brow + grp * GRP, GRP)],
                             drings[s], isems[s])

        def idx_wait(grp, s):
            pltpu.make_async_copy(src_hbm.at[pl.ds(brow + grp * GRP, GRP)],
                                  srings[s], isems[s]).wait()
            pltpu.make_async_copy(dst_hbm.at[pl.ds(brow + grp * GRP, GRP)],
                                  drings[s], isems[s]).wait()

        def gather_start(s, m, rb):
            pltpu.async_copy(y_hbm.at[srings[s].at[m]], rows[rb], gsems[rb])

        def gather_wait(s, m, rb):
            pltpu.make_async_copy(y_hbm.at[srings[s].at[m]], rows[rb],
                                  gsems[rb]).wait()

        @pl.loop(0, CHUNK)
        def _(r):
            @pl.loop(0, D, step=16)
            def _(c):
                rows0[r, pl.ds(c, 16)] = jnp.zeros((16,), jnp.float32)

        zbase = sid * ROWS_PER_TILE

        @pl.loop(0, 5 * CHUNK, step=CHUNK)
        def _(o):
            pltpu.sync_copy(rows0, acc.at[pl.ds(zbase + o, CHUNK)])

        pltpu.sync_copy(rows0.at[pl.ds(0, ROWS_PER_TILE - 5 * CHUNK)],
                        acc.at[pl.ds(zbase + 5 * CHUNK,
                                     ROWS_PER_TILE - 5 * CHUNK)])
        plsc.subcore_barrier()

        idx_start(0, 0)
        idx_wait(0, 0)
        gather_start(0, 0, 0)
        idx_start(1, 1)

        def group_body(g, s):
            for m in range(GRP):
                rb = m % 2
                gather_wait(s, m, rb)
                if m < GRP - 1:
                    gather_start(s, m + 1, 1 - rb)
                    pltpu.sync_copy(rows[rb], acc.at[drings[s].at[m]],
                                    add=True)
                else:
                    @pl.when(g + 1 < NGRP)
                    def _():
                        idx_wait(g + 1, 1 - s)
                        gather_start(1 - s, 0, 1 - rb)

                    pltpu.sync_copy(rows[rb], acc.at[drings[s].at[m]],
                                    add=True)

                    @pl.when(g + 2 < NGRP)
                    def _():
                        idx_start(g + 2, s)

        @pl.loop(0, NGRP, step=2)
        def _(g):
            group_body(g, 0)
            group_body(g + 1, 1)

        plsc.subcore_barrier()

        @pl.loop(0, ROWS_PER_TILE, step=ZROWS)
        def _(o):
            pltpu.sync_copy(acc.at[pl.ds(zbase + o, ZROWS)],
                            out_hbm.at[pl.ds(cid * NPAD + zbase + o, ZROWS)])

    return k(y, src2d, dst2d)


def _tc_matmul(x, W):
    def body(x_ref, w_ref, o_ref):
        o_ref[...] = jnp.dot(x_ref[...], w_ref[...],
                             preferred_element_type=jnp.float32)

    return pl.pallas_call(
        body,
        grid=(NBLK,),
        in_specs=[pl.BlockSpec((BLK, D), lambda i: (i, 0)),
                  pl.BlockSpec((D, D), lambda i: (0, 0))],
        out_specs=pl.BlockSpec((BLK, D), lambda i: (i, 0)),
        out_shape=jax.ShapeDtypeStruct((NPAD, D), jnp.float32),
    )(x, W)


def _tc_scale(xw, dpt):
    def body(xw_ref, dp_ref, y_ref, dinv_ref):
        deg = 1.0 + dp_ref[:, 0:1] + dp_ref[:, 1:2]
        dinv = lax.rsqrt(deg)
        dinv_ref[...] = dinv
        y_ref[...] = xw_ref[...] * dinv

    return pl.pallas_call(
        body,
        grid=(NBLK,),
        in_specs=[pl.BlockSpec((BLK, D), lambda i: (i, 0)),
                  pl.BlockSpec((BLK, 2), lambda i: (i, 0))],
        out_specs=[pl.BlockSpec((BLK, D), lambda i: (i, 0)),
                   pl.BlockSpec((BLK, 1), lambda i: (i, 0))],
        out_shape=[jax.ShapeDtypeStruct((NPAD, D), jnp.float32),
                   jax.ShapeDtypeStruct((NPAD, 1), jnp.float32)],
    )(xw, dpt)


_A0 = pl.BlockSpec((BLK, D), lambda i: (i, 0))
_A1 = pl.BlockSpec((BLK, D), lambda i: (NBLK + i, 0))


def _tc_mid(agg, y1, dinv, b1, W2):
    def body(a0_ref, a1_ref, y1_ref, dinv_ref, b1_ref, w2_ref, y2_ref):
        dinv = dinv_ref[...]
        h = (a0_ref[...] + a1_ref[...] + y1_ref[...]) * dinv + b1_ref[...]
        h = jnp.maximum(h, 0.0)
        y2_ref[...] = jnp.dot(h, w2_ref[...],
                              preferred_element_type=jnp.float32) * dinv

    return pl.pallas_call(
        body,
        grid=(NBLK,),
        in_specs=[_A0, _A1,
                  pl.BlockSpec((BLK, D), lambda i: (i, 0)),
                  pl.BlockSpec((BLK, 1), lambda i: (i, 0)),
                  pl.BlockSpec((1, D), lambda i: (0, 0)),
                  pl.BlockSpec((D, D), lambda i: (0, 0))],
        out_specs=pl.BlockSpec((BLK, D), lambda i: (i, 0)),
        out_shape=jax.ShapeDtypeStruct((NPAD, D), jnp.float32),
    )(agg, agg, y1, dinv, b1, W2)


def _tc_pool(agg, y2, dinv, b2, bcol):
    def body(a0_ref, a1_ref, y2_ref, dinv_ref, b2_ref, b_ref, p_ref):
        i = pl.program_id(0)

        @pl.when(i == 0)
        def _():
            p_ref[...] = jnp.full((N_GRAPHS, D), -jnp.inf, jnp.float32)

        h = ((a0_ref[...] + a1_ref[...] + y2_ref[...]) * dinv_ref[...]
             + b2_ref[...])
        b = b_ref[...]
        lo = jnp.maximum(jnp.min(b), 0)
        hi = jnp.max(b)

        def upd(g, carry):
            m = jnp.max(jnp.where(b == g, h, -jnp.inf), axis=0, keepdims=True)
            p_ref[pl.ds(g, 1), :] = jnp.maximum(p_ref[pl.ds(g, 1), :], m)
            return carry

        lax.fori_loop(lo, hi + 1, upd, 0)

    return pl.pallas_call(
        body,
        grid=(NBLK,),
        in_specs=[_A0, _A1,
                  pl.BlockSpec((BLK, D), lambda i: (i, 0)),
                  pl.BlockSpec((BLK, 1), lambda i: (i, 0)),
                  pl.BlockSpec((1, D), lambda i: (0, 0)),
                  pl.BlockSpec((BLK, 1), lambda i: (i, 0))],
        out_specs=pl.BlockSpec((N_GRAPHS, D), lambda i: (0, 0)),
        out_shape=jax.ShapeDtypeStruct((N_GRAPHS, D), jnp.float32),
    )(agg, agg, y2, dinv, b2, bcol)


def _tc_dec(pooled, Wd, bd):
    NP = Wd.shape[1]
    CBLK = 1024

    def body(p_ref, wd_ref, bd_ref, o_ref):
        o_ref[...] = jnp.dot(p_ref[...], wd_ref[...],
                             preferred_element_type=jnp.float32) + bd_ref[...]

    return pl.pallas_call(
        body,
        grid=(NP // CBLK,),
        in_specs=[pl.BlockSpec((N_GRAPHS, D), lambda i: (0, 0)),
                  pl.BlockSpec((D, CBLK), lambda i: (0, i)),
                  pl.BlockSpec((1, CBLK), lambda i: (0, i))],
        out_specs=pl.BlockSpec((N_GRAPHS, CBLK), lambda i: (0, i)),
        out_shape=jax.ShapeDtypeStruct((N_GRAPHS, NP), jnp.float32),
    )(pooled, Wd, bd)


def kernel(x, edge_index, batch, W1, b1, W2, b2, Wd, bd):
    src2d = edge_index[0].reshape(N_EDGES // CHUNK, CHUNK)
    dst2d = edge_index[1].reshape(N_EDGES // CHUNK, CHUNK)

    x_p = jnp.pad(x, ((0, NPAD - N_NODES), (0, 0)))
    b_p = jnp.pad(batch.reshape(N_NODES, 1), ((0, NPAD - N_NODES), (0, 0)),
                  constant_values=-1)

    degp = _sc_degree(dst2d).reshape(NC, HP)
    xw1 = _tc_matmul(x_p, W1)
    dpt = jnp.transpose(degp)
    y1, dinv = _tc_scale(xw1, dpt)

    agg1 = _sc_aggregate(y1, src2d, dst2d)
    y2 = _tc_mid(agg1, y1, dinv, b1.reshape(1, D), W2)

    agg2 = _sc_aggregate(y2, src2d, dst2d)
    pooled = _tc_pool(agg2, y2, dinv, b2.reshape(1, D), b_p)

    Wd_p = jnp.pad(Wd, ((0, 0), (0, NPAD - N_NODES)))
    bd_p = jnp.pad(bd.reshape(1, N_NODES), ((0, 0), (0, NPAD - N_NODES)))
    return _tc_dec(pooled, Wd_p, bd_p)[:, :N_NODES]
